# baseline (speedup 1.0000x reference)
import jax
import jax.numpy as jnp
from jax import lax
from jax.experimental import pallas as pl
from jax.experimental.pallas import tpu as pltpu
from jax.experimental.pallas import tpu_sc as plsc

_A = 5
_C = 80
_G = 20
_GY = 19
_GX = 19
_P = _GY * _GX
_L = 16
_CH = 23
_NCH = _A * _CH
_PP = _CH * _L
_IOU_T = 0.6
_L_OBJ = 5.0
_L_PRIOR = 0.01
_EPS = 1e-5
_BIG_N = 2 ** 30


def _sc_match_body(gtb_hbm, gtl_hbm, anch_hbm, tgt_hbm,
                   gtb, gtl, anch, db, bo, bidx, tgt, tgtp):
    ncores = 2
    wid = lax.axis_index("s") * ncores + lax.axis_index("c")

    pltpu.sync_copy(gtb_hbm, gtb)
    pltpu.sync_copy(gtl_hbm, gtl)
    pltpu.sync_copy(anch_hbm, anch)

    lane = lax.iota(jnp.int32, 16)
    zero_i = lane * 0
    widv = zero_i + wid

    def fill(c, _):
        a = c // _CH
        j = c - a * _CH
        pos = j * _L + lane
        valid = pos < _P
        ys = (pos.astype(jnp.float32) / float(_GX)).astype(jnp.int32)
        xs = pos - ys * _GX
        cx = (xs.astype(jnp.float32) + 0.5) / float(_GX)
        cy = (ys.astype(jnp.float32) + 0.5) / float(_GY)
        aw = plsc.load_gather(anch, [zero_i + a, zero_i])
        ah = plsc.load_gather(anch, [zero_i + a, zero_i + 1])
        x1 = jnp.where(valid, cx - aw * 0.5, 0.0)
        y1 = jnp.where(valid, cy - ah * 0.5, 0.0)
        x2 = jnp.where(valid, cx + aw * 0.5, 0.0)
        y2 = jnp.where(valid, cy + ah * 0.5, 0.0)
        db[0, c] = x1
        db[1, c] = y1
        db[2, c] = x2
        db[3, c] = y2
        db[4, c] = (x2 - x1) * (y2 - y1)
        bo[c] = x1 * 0.0
        bidx[c] = zero_i
        return 0

    lax.fori_loop(0, _NCH, fill, 0)

    def gt_splat(g):
        gv = zero_i + g
        cxs = plsc.load_gather(gtb, [widv, gv, zero_i])
        cys = plsc.load_gather(gtb, [widv, gv, zero_i + 1])
        ws = plsc.load_gather(gtb, [widv, gv, zero_i + 2])
        hs = plsc.load_gather(gtb, [widv, gv, zero_i + 3])
        gx1 = cxs - ws * 0.5
        gy1 = cys - hs * 0.5
        gx2 = cxs + ws * 0.5
        gy2 = cys + hs * 0.5
        gar = (gx2 - gx1) * (gy2 - gy1)
        return gx1, gy1, gx2, gy2, gar

    def iou_chunk(gt, x1, y1, x2, y2, ar):
        gx1, gy1, gx2, gy2, gar = gt
        iw = jnp.maximum(jnp.minimum(gx2, x2) - jnp.maximum(gx1, x1), 0.0)
        ih = jnp.maximum(jnp.minimum(gy2, y2) - jnp.maximum(gy1, y1), 0.0)
        inter = iw * ih
        return inter / (gar + ar - inter + _EPS)

    nstar = []
    for g in range(0, _G, 2):
        gt0 = gt_splat(g)
        gt1 = gt_splat(g + 1)

        def body(c, carry, gt0=gt0, gt1=gt1, g=g):
            pm0, pa0, pm1, pa1 = carry
            x1 = db[0, c]
            y1 = db[1, c]
            x2 = db[2, c]
            y2 = db[3, c]
            ar = db[4, c]
            nv = (c * _L - 7 * (c // _CH)) + lane
            iou0 = iou_chunk(gt0, x1, y1, x2, y2, ar)
            iou1 = iou_chunk(gt1, x1, y1, x2, y2, ar)
            o = bo[c]
            bi = bidx[c]
            b0 = iou0 > o
            o = jnp.where(b0, iou0, o)
            bi = jnp.where(b0, zero_i + g, bi)
            b1 = iou1 > o
            bo[c] = jnp.where(b1, iou1, o)
            bidx[c] = jnp.where(b1, zero_i + (g + 1), bi)
            p0 = iou0 > pm0
            p1 = iou1 > pm1
            return (jnp.where(p0, iou0, pm0), jnp.where(p0, nv, pa0),
                    jnp.where(p1, iou1, pm1), jnp.where(p1, nv, pa1))

        init = (jnp.full((_L,), -1.0, jnp.float32), jnp.zeros((_L,), jnp.int32))
        pm0, pa0, pm1, pa1 = lax.fori_loop(0, _NCH, body, init + init)

        for pm, pa in ((pm0, pa0), (pm1, pa1)):
            m = jnp.max(pm)
            cand = jnp.where(pm == m, pa, _BIG_N)
            nstar.append(jnp.min(cand))

    def thr(c, _):
        a = c // _CH
        j = c - a * _CH
        over = bo[c] > _IOU_T
        bi = bidx[c]
        sl = pl.ds(j * _L, _L)
        for f in range(4):
            v = plsc.load_gather(gtb, [widv, bi, zero_i + f])
            tgt[a, f, sl] = jnp.where(over, v, 0.0)
        vl = plsc.load_gather(gtl, [widv, bi]).astype(jnp.float32)
        tgt[a, 4, sl] = jnp.where(over, vl, 0.0)
        return 0

    lax.fori_loop(0, _NCH, thr, 0)

    for g in range(_G):
        gv = zero_i + g
        n = nstar[g]
        a = n // _P
        pos = n - a * _P
        j = pos // _L
        ll = pos - j * _L
        hit = lane == ll
        sl = pl.ds(j * _L, _L)
        for f in range(4):
            v = plsc.load_gather(gtb, [widv, gv, zero_i + f])
            tgt[a, f, sl] = jnp.where(hit, v, tgt[a, f, sl])
        vl = plsc.load_gather(gtl, [widv, gv]).astype(jnp.float32)
        tgt[a, 4, sl] = jnp.where(hit, vl, tgt[a, 4, sl])

    def relay(r, _):
        af = r // _GY
        ys = r - af * _GY
        a = af // 5
        f = af - a * 5
        base = ys * _GX
        v0 = tgt[a, f, pl.ds(base, _L)]
        v1 = tgt[a, f, pl.ds(base + _L, _L)]
        tgtp[a, f, ys, pl.ds(0, _L)] = v0
        tgtp[a, f, ys, pl.ds(_L, _L)] = jnp.where(lane + _L < _GX, v1, 0.0)
        return 0

    lax.fori_loop(0, _A * 5 * _GY, relay, 0)

    pltpu.sync_copy(tgtp, tgt_hbm.at[wid])


def _sc_match(gt_boxes, gt_labels, anchors, batch):
    kern = pl.kernel(
        _sc_match_body,
        out_type=jax.ShapeDtypeStruct((batch, _A, 5, 24, 128), jnp.float32),
        mesh=plsc.VectorSubcoreMesh(core_axis_name="c", subcore_axis_name="s"),
        scratch_types=[
            pltpu.VMEM((batch, _G, 4), jnp.float32),
            pltpu.VMEM((batch, _G), jnp.int32),
            pltpu.VMEM((_A, 2), jnp.float32),
            pltpu.VMEM((5, _NCH, _L), jnp.float32),
            pltpu.VMEM((_NCH, _L), jnp.float32),
            pltpu.VMEM((_NCH, _L), jnp.int32),
            pltpu.VMEM((_A, 5, _PP + _L), jnp.float32),
            pltpu.VMEM((_A, 5, 24, 128), jnp.float32),
        ],
        compiler_params=pltpu.CompilerParams(use_tc_tiling_on_sc=False,
                                             needs_layout_passes=False),
    )
    return kern(gt_boxes, gt_labels, anchors)


def _tc_loss_body(seen_ref, anch_ref, p_ref, t_ref, o_ref):
    inv = 0.5 / float(_GX)
    noobj_l = 0.0
    obj_l = 0.0
    coord_l = 0.0
    cls_l = 0.0
    prior_l = 0.0
    for a in range(_A):
        base = a * (_C + 5)
        t = t_ref[0, a]
        aw = anch_ref[a, 0]
        ah = anch_ref[a, 1]
        x = jax.nn.sigmoid(p_ref[0, base + 0])
        y = jax.nn.sigmoid(p_ref[0, base + 1])
        w = jnp.exp(p_ref[0, base + 2]) * aw
        h = jnp.exp(p_ref[0, base + 3]) * ah
        obj = jax.nn.sigmoid(p_ref[0, base + 4])
        cls = p_ref[0, pl.ds(base + 5, _C)]

        tx = t[0, 0:_GY, 0:_GX]
        ty = t[1, 0:_GY, 0:_GX]
        tw = t[2, 0:_GY, 0:_GX]
        th = t[3, 0:_GY, 0:_GX]
        lab = t[4, 0:_GY, 0:_GX]
        pos = (lab > 0).astype(jnp.float32)

        ax1 = x - w * 0.5
        ay1 = y - h * 0.5
        ax2 = x + w * 0.5
        ay2 = y + h * 0.5
        bx1 = tx - tw * 0.5
        by1 = ty - th * 0.5
        bx2 = tx + tw * 0.5
        by2 = ty + th * 0.5
        iw = jnp.maximum(jnp.minimum(ax2, bx2) - jnp.maximum(ax1, bx1), 0.0)
        ih = jnp.maximum(jnp.minimum(ay2, by2) - jnp.maximum(ay1, by1), 0.0)
        inter = iw * ih
        area_a = (ax2 - ax1) * (ay2 - ay1)
        area_b = (bx2 - bx1) * (by2 - by1)
        iou = inter / (area_a + area_b - inter + _EPS)

        noobj = (iou < _IOU_T).astype(jnp.float32)
        noobj_l += jnp.sum((noobj * obj) ** 2)
        obj_l += jnp.sum((pos * obj - pos * iou) ** 2)
        coord_l += (jnp.sum((pos * x - tx) ** 2) + jnp.sum((pos * y - ty) ** 2)
                    + jnp.sum((pos * w - tw) ** 2) + jnp.sum((pos * h - th) ** 2))

        pc = pos[None] * cls
        mx = jnp.max(pc, axis=0, keepdims=True)
        e = jnp.exp(pc - lax.stop_gradient(mx))
        sm = e / jnp.sum(e, axis=0, keepdims=True)
        labi = lab.astype(jnp.int32)[None]
        oh = (lax.broadcasted_iota(jnp.int32, (_C, _GY, _GX), 0) == labi)
        cls_l += jnp.sum((sm - oh.astype(jnp.float32)) ** 2)

        neg = 1.0 - pos
        prior_l += (jnp.sum((neg * x - neg * inv) ** 2)
                    + jnp.sum((neg * y - neg * inv) ** 2)
                    + jnp.sum((neg * w - neg * aw) ** 2)
                    + jnp.sum((neg * h - neg * ah) ** 2))

    pfac = jnp.where(seen_ref[0] < 12800, _L_PRIOR, 0.0)
    total = cls_l + noobj_l + _L_OBJ * obj_l + coord_l + pfac * prior_l
    o_ref[0] = jnp.full((1, 1), total, jnp.float32)


def _tc_loss(pred, tgt, anchors, seen_arr, interpret=False):
    batch = pred.shape[0]
    return pl.pallas_call(
        _tc_loss_body,
        grid=(batch,),
        in_specs=[
            pl.BlockSpec(memory_space=pltpu.SMEM),
            pl.BlockSpec(memory_space=pltpu.SMEM),
            pl.BlockSpec((1, _A * (_C + 5), _GY, _GX), lambda b: (b, 0, 0, 0)),
            pl.BlockSpec((1, _A, 5, 24, 128), lambda b: (b, 0, 0, 0, 0)),
        ],
        out_specs=pl.BlockSpec((1, 1, 1), lambda b: (b, 0, 0)),
        out_shape=jax.ShapeDtypeStruct((batch, 1, 1), jnp.float32),
        interpret=interpret,
    )(seen_arr, anchors, pred, tgt)


def kernel(prediction, gt_boxes, gt_labels, anchors, seen):
    batch, ch, gy, gx = prediction.shape
    anchors = anchors.astype(jnp.float32)
    lab = gt_labels.astype(jnp.int32)

    tgt = _sc_match(gt_boxes, lab, anchors, batch)
    seen_arr = jnp.asarray(seen, jnp.int32).reshape(1)
    partial = _tc_loss(prediction, tgt, anchors, seen_arr)
    return jnp.sum(partial)


# speedup vs baseline: 1.7437x; 1.7437x over previous
import jax
import jax.numpy as jnp
from jax import lax
from jax.experimental import pallas as pl
from jax.experimental.pallas import tpu as pltpu
from jax.experimental.pallas import tpu_sc as plsc

_A = 5
_C = 80
_G = 20
_GY = 19
_GX = 19
_P = _GY * _GX
_L = 16
_CH = 23
_NCH = _A * _CH
_PP = _CH * _L
_IOU_T = 0.6
_L_OBJ = 5.0
_L_PRIOR = 0.01
_EPS = 1e-5
_BIG_N = 2 ** 30


def _sc_match_body(gtb_hbm, gtl_hbm, anch_hbm, tgt_hbm,
                   gtb, gtl, anch, db, bo, bidx, tgt):
    ncores = 2
    wid = lax.axis_index("s") * ncores + lax.axis_index("c")

    pltpu.sync_copy(gtb_hbm, gtb)
    pltpu.sync_copy(gtl_hbm, gtl)
    pltpu.sync_copy(anch_hbm, anch)

    lane = lax.iota(jnp.int32, 16)
    zero_i = lane * 0
    widv = zero_i + wid

    def fill(c, _):
        a = c // _CH
        j = c - a * _CH
        pos = j * _L + lane
        valid = pos < _P
        ys = (pos.astype(jnp.float32) / float(_GX)).astype(jnp.int32)
        xs = pos - ys * _GX
        cx = (xs.astype(jnp.float32) + 0.5) / float(_GX)
        cy = (ys.astype(jnp.float32) + 0.5) / float(_GY)
        aw = plsc.load_gather(anch, [zero_i + a, zero_i])
        ah = plsc.load_gather(anch, [zero_i + a, zero_i + 1])
        x1 = jnp.where(valid, cx - aw * 0.5, 0.0)
        y1 = jnp.where(valid, cy - ah * 0.5, 0.0)
        x2 = jnp.where(valid, cx + aw * 0.5, 0.0)
        y2 = jnp.where(valid, cy + ah * 0.5, 0.0)
        db[0, c] = x1
        db[1, c] = y1
        db[2, c] = x2
        db[3, c] = y2
        db[4, c] = (x2 - x1) * (y2 - y1)
        bo[c] = x1 * 0.0
        bidx[c] = zero_i
        return 0

    lax.fori_loop(0, _NCH, fill, 0)

    def gt_splat(g):
        gv = zero_i + g
        cxs = plsc.load_gather(gtb, [widv, gv, zero_i])
        cys = plsc.load_gather(gtb, [widv, gv, zero_i + 1])
        ws = plsc.load_gather(gtb, [widv, gv, zero_i + 2])
        hs = plsc.load_gather(gtb, [widv, gv, zero_i + 3])
        gx1 = cxs - ws * 0.5
        gy1 = cys - hs * 0.5
        gx2 = cxs + ws * 0.5
        gy2 = cys + hs * 0.5
        gar = (gx2 - gx1) * (gy2 - gy1)
        return gx1, gy1, gx2, gy2, gar

    def iou_chunk(gt, x1, y1, x2, y2, ar):
        gx1, gy1, gx2, gy2, gar = gt
        iw = jnp.maximum(jnp.minimum(gx2, x2) - jnp.maximum(gx1, x1), 0.0)
        ih = jnp.maximum(jnp.minimum(gy2, y2) - jnp.maximum(gy1, y1), 0.0)
        inter = iw * ih
        return inter / (gar + ar - inter + _EPS)

    nstar = []
    for g in range(0, _G, 2):
        gt0 = gt_splat(g)
        gt1 = gt_splat(g + 1)

        def body(c, carry, gt0=gt0, gt1=gt1, g=g):
            pm0, pa0, pm1, pa1 = carry
            x1 = db[0, c]
            y1 = db[1, c]
            x2 = db[2, c]
            y2 = db[3, c]
            ar = db[4, c]
            nv = (c * _L - 7 * (c // _CH)) + lane
            iou0 = iou_chunk(gt0, x1, y1, x2, y2, ar)
            iou1 = iou_chunk(gt1, x1, y1, x2, y2, ar)
            o = bo[c]
            bi = bidx[c]
            b0 = iou0 > o
            o = jnp.where(b0, iou0, o)
            bi = jnp.where(b0, zero_i + g, bi)
            b1 = iou1 > o
            bo[c] = jnp.where(b1, iou1, o)
            bidx[c] = jnp.where(b1, zero_i + (g + 1), bi)
            p0 = iou0 > pm0
            p1 = iou1 > pm1
            return (jnp.where(p0, iou0, pm0), jnp.where(p0, nv, pa0),
                    jnp.where(p1, iou1, pm1), jnp.where(p1, nv, pa1))

        init = (jnp.full((_L,), -1.0, jnp.float32), jnp.zeros((_L,), jnp.int32))
        pm0, pa0, pm1, pa1 = lax.fori_loop(0, _NCH, body, init + init)

        for pm, pa in ((pm0, pa0), (pm1, pa1)):
            m = jnp.max(pm)
            cand = jnp.where(pm == m, pa, _BIG_N)
            nstar.append(jnp.min(cand))

    def thr(c, _):
        a = c // _CH
        j = c - a * _CH
        over = bo[c] > _IOU_T
        bi = bidx[c]
        sl = pl.ds(j * _L, _L)
        for f in range(4):
            v = plsc.load_gather(gtb, [widv, bi, zero_i + f])
            tgt[a, f, sl] = jnp.where(over, v, 0.0)
        vl = plsc.load_gather(gtl, [widv, bi]).astype(jnp.float32)
        tgt[a, 4, sl] = jnp.where(over, vl, 0.0)
        return 0

    lax.fori_loop(0, _NCH, thr, 0)

    for g in range(_G):
        gv = zero_i + g
        n = nstar[g]
        a = n // _P
        pos = n - a * _P
        j = pos // _L
        ll = pos - j * _L
        hit = lane == ll
        sl = pl.ds(j * _L, _L)
        for f in range(4):
            v = plsc.load_gather(gtb, [widv, gv, zero_i + f])
            tgt[a, f, sl] = jnp.where(hit, v, tgt[a, f, sl])
        vl = plsc.load_gather(gtl, [widv, gv]).astype(jnp.float32)
        tgt[a, 4, sl] = jnp.where(hit, vl, tgt[a, 4, sl])

    pltpu.sync_copy(tgt, tgt_hbm.at[wid])


def _sc_match(gt_boxes, gt_labels, anchors, batch):
    kern = pl.kernel(
        _sc_match_body,
        out_type=jax.ShapeDtypeStruct((batch, _A, 5, _PP), jnp.float32),
        mesh=plsc.VectorSubcoreMesh(core_axis_name="c", subcore_axis_name="s"),
        scratch_types=[
            pltpu.VMEM((batch, _G, 4), jnp.float32),
            pltpu.VMEM((batch, _G), jnp.int32),
            pltpu.VMEM((_A, 2), jnp.float32),
            pltpu.VMEM((5, _NCH, _L), jnp.float32),
            pltpu.VMEM((_NCH, _L), jnp.float32),
            pltpu.VMEM((_NCH, _L), jnp.int32),
            pltpu.VMEM((_A, 5, _PP), jnp.float32),
        ],
        compiler_params=pltpu.CompilerParams(use_tc_tiling_on_sc=False,
                                             needs_layout_passes=False),
    )
    return kern(gt_boxes, gt_labels, anchors)


def _tc_loss_body(seen_ref, anch_ref, p_ref, t_ref, o_ref):
    inv = 0.5 / float(_GX)
    noobj_l = 0.0
    obj_l = 0.0
    coord_l = 0.0
    cls_l = 0.0
    prior_l = 0.0
    for a in range(_A):
        blk = p_ref[0, a]
        t = t_ref[0, a]
        aw = anch_ref[a, 0]
        ah = anch_ref[a, 1]
        x = jax.nn.sigmoid(blk[0:1, :])
        y = jax.nn.sigmoid(blk[1:2, :])
        w = jnp.exp(blk[2:3, :]) * aw
        h = jnp.exp(blk[3:4, :]) * ah
        obj = jax.nn.sigmoid(blk[4:5, :])
        cls = blk[5:, :]

        tx = t[0:1, :_P]
        ty = t[1:2, :_P]
        tw = t[2:3, :_P]
        th = t[3:4, :_P]
        lab = t[4:5, :_P]
        pos = (lab > 0).astype(jnp.float32)

        ax1 = x - w * 0.5
        ay1 = y - h * 0.5
        ax2 = x + w * 0.5
        ay2 = y + h * 0.5
        bx1 = tx - tw * 0.5
        by1 = ty - th * 0.5
        bx2 = tx + tw * 0.5
        by2 = ty + th * 0.5
        iw = jnp.maximum(jnp.minimum(ax2, bx2) - jnp.maximum(ax1, bx1), 0.0)
        ih = jnp.maximum(jnp.minimum(ay2, by2) - jnp.maximum(ay1, by1), 0.0)
        inter = iw * ih
        area_a = (ax2 - ax1) * (ay2 - ay1)
        area_b = (bx2 - bx1) * (by2 - by1)
        iou = inter / (area_a + area_b - inter + _EPS)

        noobj = (iou < _IOU_T).astype(jnp.float32)
        noobj_l += jnp.sum((noobj * obj) ** 2)
        obj_l += jnp.sum((pos * obj - pos * iou) ** 2)
        coord_l += (jnp.sum((pos * x - tx) ** 2) + jnp.sum((pos * y - ty) ** 2)
                    + jnp.sum((pos * w - tw) ** 2) + jnp.sum((pos * h - th) ** 2))

        pc = pos * cls
        mx = jnp.max(pc, axis=0, keepdims=True)
        e = jnp.exp(pc - lax.stop_gradient(mx))
        sm = e / jnp.sum(e, axis=0, keepdims=True)
        labi = lab.astype(jnp.int32)
        oh = (lax.broadcasted_iota(jnp.int32, (_C, _P), 0) == labi)
        cls_l += jnp.sum((sm - oh.astype(jnp.float32)) ** 2)

        neg = 1.0 - pos
        prior_l += (jnp.sum((neg * x - neg * inv) ** 2)
                    + jnp.sum((neg * y - neg * inv) ** 2)
                    + jnp.sum((neg * w - neg * aw) ** 2)
                    + jnp.sum((neg * h - neg * ah) ** 2))

    pfac = jnp.where(seen_ref[0] < 12800, _L_PRIOR, 0.0)
    total = cls_l + noobj_l + _L_OBJ * obj_l + coord_l + pfac * prior_l

    @pl.when(pl.program_id(0) == 0)
    def _init():
        o_ref[0] = jnp.zeros((1, 1), jnp.float32)

    o_ref[0] = o_ref[0] + jnp.full((1, 1), total, jnp.float32)


def _tc_loss(pred, tgt, anchors, seen_arr, interpret=False):
    batch = pred.shape[0]
    return pl.pallas_call(
        _tc_loss_body,
        grid=(batch,),
        in_specs=[
            pl.BlockSpec(memory_space=pltpu.SMEM),
            pl.BlockSpec(memory_space=pltpu.SMEM),
            pl.BlockSpec((1, _A, _C + 5, _P), lambda b: (b, 0, 0, 0)),
            pl.BlockSpec((1, _A, 5, _PP), lambda b: (b, 0, 0, 0)),
        ],
        out_specs=pl.BlockSpec((1, 1, 1), lambda b: (0, 0, 0)),
        out_shape=jax.ShapeDtypeStruct((1, 1, 1), jnp.float32),
        interpret=interpret,
    )(seen_arr, anchors, pred, tgt)


def kernel(prediction, gt_boxes, gt_labels, anchors, seen):
    batch, ch, gy, gx = prediction.shape
    anchors = anchors.astype(jnp.float32)
    lab = gt_labels.astype(jnp.int32)

    tgt = _sc_match(gt_boxes, lab, anchors, batch)
    pred = prediction.reshape(batch, _A, ch // _A, gy * gx)
    seen_arr = jnp.asarray(seen, jnp.int32).reshape(1)
    partial = _tc_loss(pred, tgt, anchors, seen_arr)
    return partial[0, 0, 0]
